# baseline (speedup 1.0000x reference)
import functools

import jax
import jax.numpy as jnp
from jax import lax
from jax.experimental import pallas as pl
from jax.experimental.pallas import tpu as pltpu
from jax.experimental.pallas import tpu_sc as plsc

NUM_EMB = 1_000_000
EMB_DIM = 64
N_IDX = 4096 * 50
QMAX = 127.0

_RED_BLK = 8000


def _absmax_body(w_ref, out_ref):
    i = pl.program_id(0)
    m = jnp.max(jnp.abs(w_ref[...]))
    prev = jnp.where(i == 0, 0.0, out_ref[0, 0])
    out_ref[0, 0] = jnp.maximum(prev, m)

    @pl.when(i == pl.num_programs(0) - 1)
    def _():
        out_ref[0, 0] = jnp.maximum(out_ref[0, 0], 1e-8) / QMAX


def _scale_of(weight):
    return pl.pallas_call(
        _absmax_body,
        grid=(NUM_EMB // _RED_BLK,),
        in_specs=[pl.BlockSpec((_RED_BLK, EMB_DIM), lambda i: (i, 0))],
        out_specs=pl.BlockSpec(memory_space=pltpu.SMEM),
        out_shape=jax.ShapeDtypeStruct((1, 1), jnp.float32),
    )(weight)


_NC, _NS = 2, 16
_NW = _NC * _NS
_B_PER_W = N_IDX // _NW
_CHUNK = 128
_N_CHUNKS = _B_PER_W // _CHUNK


def _sc_gather_body(table_hbm, idx_hbm, out_hbm, idx_v, rows_v, sem):
    wid = lax.axis_index("s") * _NC + lax.axis_index("c")
    base = wid * _B_PER_W
    pltpu.sync_copy(idx_hbm.at[pl.ds(base, _B_PER_W)], idx_v)

    def chunk(c, carry):
        off = c * _CHUNK
        pltpu.async_copy(
            table_hbm.at[idx_v.at[pl.ds(off, _CHUNK)]], rows_v, sem
        ).wait()
        pltpu.sync_copy(rows_v, out_hbm.at[pl.ds(base + off, _CHUNK)])
        return carry

    lax.fori_loop(0, _N_CHUNKS, chunk, 0)


def _sc_gather(weight, idx):
    mesh = plsc.VectorSubcoreMesh(
        core_axis_name="c", subcore_axis_name="s",
        num_cores=_NC, num_subcores=_NS,
    )
    fn = functools.partial(
        pl.kernel,
        mesh=mesh,
        out_type=jax.ShapeDtypeStruct((N_IDX, EMB_DIM), jnp.float32),
        scratch_types=[
            pltpu.VMEM((_B_PER_W,), jnp.int32),
            pltpu.VMEM((_CHUNK, EMB_DIM), jnp.float32),
            pltpu.SemaphoreType.DMA,
        ],
        compiler_params=pltpu.CompilerParams(use_tc_tiling_on_sc=False),
    )(_sc_gather_body)
    return fn(weight, idx)


_Q_ROWS = N_IDX * EMB_DIM // 128
_Q_BLK = 2048


def _quant_body(scale_ref, g_ref, out_ref):
    inv = 1.0 / scale_ref[0, 0]
    q = jnp.round(g_ref[...] * inv)
    out_ref[...] = jnp.clip(q, -QMAX, QMAX - 1.0).astype(jnp.int8)


def _quantize(gathered2d, scale):
    return pl.pallas_call(
        _quant_body,
        grid=(_Q_ROWS // _Q_BLK,),
        in_specs=[
            pl.BlockSpec(memory_space=pltpu.SMEM),
            pl.BlockSpec((_Q_BLK, 128), lambda i: (i, 0)),
        ],
        out_specs=pl.BlockSpec((_Q_BLK, 128), lambda i: (i, 0)),
        out_shape=jax.ShapeDtypeStruct((_Q_ROWS, 128), jnp.int8),
    )(scale, gathered2d)


def kernel(x, weight):
    scale = _scale_of(weight)
    idx = x.reshape(-1)
    gathered = _sc_gather(weight, idx)
    q8 = _quantize(gathered.reshape(_Q_ROWS, 128), scale)
    emb_int = q8.reshape(4096, 50, EMB_DIM)
    return emb_int, scale.reshape(1)
